# baseline (speedup 1.0000x reference)
import functools

import jax
import jax.numpy as jnp
from jax import lax
from jax.experimental import pallas as pl
from jax.experimental.pallas import tpu as pltpu
from jax.experimental.pallas import tpu_sc as plsc

NW = 32
BC = 112
BN = 3584


def _mm_body(k, x_ref, w_ref, b_ref, o_ref):
    x = x_ref[...]
    for j in range(k):
        y = jnp.dot(x, w_ref[j], preferred_element_type=jnp.float32)
        if j == 0:
            y = y + b_ref[...]
        o_ref[j] = y


def _matmul_tables(x, wk, bias):
    npad, cin = x.shape
    k, _, oc = wk.shape
    nb = npad // BN
    return pl.pallas_call(
        functools.partial(_mm_body, k),
        grid=(nb,),
        in_specs=[
            pl.BlockSpec((BN, cin), lambda i: (i, 0)),
            pl.BlockSpec((k, cin, oc), lambda i: (0, 0, 0)),
            pl.BlockSpec((1, oc), lambda i: (0, 0)),
        ],
        out_specs=pl.BlockSpec((k, BN, oc), lambda i: (0, i, 0)),
        out_shape=jax.ShapeDtypeStruct((k, npad, oc), jnp.float32),
        compiler_params=pltpu.CompilerParams(
            dimension_semantics=("parallel",)
        ),
    )(x, wk, bias)


def _gather_sum(idxc, table, oc, act, npad):
    nch = idxc.shape[1]
    kk = idxc.shape[2]
    mesh = plsc.VectorSubcoreMesh(core_axis_name="c", subcore_axis_name="s")

    @functools.partial(
        pl.kernel,
        out_type=jax.ShapeDtypeStruct((npad, oc), jnp.float32),
        mesh=mesh,
        scratch_types=(
            [pltpu.VMEM((kk, BC), jnp.int32)]
            + [pltpu.VMEM((BC, oc), jnp.float32) for _ in range(kk)]
            + [pltpu.VMEM((BC, oc), jnp.float32), pltpu.SemaphoreType.DMA]
        ),
        compiler_params=pltpu.CompilerParams(use_tc_tiling_on_sc=False),
    )
    def run(idx_hbm, table_hbm, out_hbm, idx_v, *rest):
        bufs = rest[:kk]
        out_v = rest[kk]
        sem = rest[kk + 1]
        wid = lax.axis_index("s") * 2 + lax.axis_index("c")
        base0 = wid * (nch * BC)
        for c in range(nch):
            base = base0 + c * BC
            pltpu.sync_copy(idx_hbm.at[wid, c], idx_v)
            cps = [
                pltpu.async_copy(table_hbm.at[idx_v.at[j]], bufs[j], sem)
                for j in range(kk)
            ]
            for cp in cps:
                cp.wait()

            def row(r, carry):
                for c2 in range(oc // 16):
                    sl = pl.ds(c2 * 16, 16)
                    s = bufs[0][r, sl]
                    for j in range(1, kk):
                        s = s + bufs[j][r, sl]
                    if act:
                        s = 1.0 / (1.0 + jnp.exp(-s))
                    out_v[r, sl] = s
                return carry

            lax.fori_loop(0, BC, row, 0)
            pltpu.sync_copy(out_v, out_hbm.at[pl.ds(base, BC)])

    return run(idxc, table)


def kernel(features, knn_indices, W0, b0, W1, b1, W2, b2):
    n, _ = features.shape
    k = knn_indices.shape[1]
    nch = -(-n // (NW * BC))
    npad = NW * BC * nch

    x = jnp.pad(features, ((0, npad - n), (0, 0)))
    idx = jnp.pad(knn_indices, ((0, npad - n), (0, 0)))
    idxa = idx + (jnp.arange(k, dtype=jnp.int32) * npad)[None, :]
    idxc = idxa.reshape(NW, nch, BC, k).transpose(0, 1, 3, 2)

    w2p = jnp.pad(W2, ((0, 16 - W2.shape[0]), (0, 0)))
    b2p = jnp.pad(b2, ((0, 0), (0, 16 - b2.shape[1])))

    h = x
    for wgt, bias, act in ((W0, b0, True), (W1, b1, True), (w2p, b2p, False)):
        oc = wgt.shape[0]
        cin = h.shape[1]
        wk = wgt.reshape(oc, k, cin).transpose(1, 2, 0)
        y = _matmul_tables(h, wk, bias)
        table = y.reshape(npad * k, oc)
        h = _gather_sum(idxc, table, oc, act, npad)

    return h[:n, :3]


# speedup vs baseline: 2.1771x; 2.1771x over previous
import functools

import jax
import jax.numpy as jnp
from jax import lax
from jax.experimental import pallas as pl
from jax.experimental.pallas import tpu as pltpu
from jax.experimental.pallas import tpu_sc as plsc

NW = 32
BC = 112
BN = 3584


def _mm_body(nj, x_ref, w_ref, b_ref, o_ref):
    x = x_ref[...]
    for j in range(nj):
        y = jnp.dot(x, w_ref[j], preferred_element_type=jnp.float32)
        if j == 0:
            y = y + b_ref[...]
        o_ref[j] = y


def _matmul_tables(x, wgrp, brow):
    npad, cin = x.shape
    nj = wgrp.shape[0]
    nb = npad // BN
    return pl.pallas_call(
        functools.partial(_mm_body, nj),
        grid=(nb,),
        in_specs=[
            pl.BlockSpec((BN, cin), lambda i: (i, 0)),
            pl.BlockSpec((nj, cin, 128), lambda i: (0, 0, 0)),
            pl.BlockSpec((1, 128), lambda i: (0, 0)),
        ],
        out_specs=pl.BlockSpec((nj, BN, 128), lambda i: (0, i, 0)),
        out_shape=jax.ShapeDtypeStruct((nj, npad, 128), jnp.float32),
        compiler_params=pltpu.CompilerParams(
            dimension_semantics=("parallel",)
        ),
    )(x, wgrp, brow)


def _gather_sum(idxc, table, oc, act, npad):
    nch = idxc.shape[1]
    kk = idxc.shape[2]
    mesh = plsc.VectorSubcoreMesh(core_axis_name="c", subcore_axis_name="s")

    @functools.partial(
        pl.kernel,
        out_type=jax.ShapeDtypeStruct((npad, oc), jnp.float32),
        mesh=mesh,
        scratch_types=(
            [pltpu.VMEM((kk, BC), jnp.int32)]
            + [pltpu.VMEM((BC, oc), jnp.float32) for _ in range(kk)]
            + [pltpu.VMEM((BC, oc), jnp.float32), pltpu.SemaphoreType.DMA]
        ),
        compiler_params=pltpu.CompilerParams(use_tc_tiling_on_sc=False),
    )
    def run(idx_hbm, table_hbm, out_hbm, idx_v, *rest):
        bufs = rest[:kk]
        out_v = rest[kk]
        sem = rest[kk + 1]
        wid = lax.axis_index("s") * 2 + lax.axis_index("c")
        base0 = wid * (nch * BC)
        for c in range(nch):
            base = base0 + c * BC
            pltpu.sync_copy(idx_hbm.at[wid, c], idx_v)
            cps = [
                pltpu.async_copy(table_hbm.at[idx_v.at[j]], bufs[j], sem)
                for j in range(kk)
            ]
            for cp in cps:
                cp.wait()

            def row(r, carry):
                for c2 in range(oc // 16):
                    sl = pl.ds(c2 * 16, 16)
                    s = bufs[0][r, sl]
                    for j in range(1, kk):
                        s = s + bufs[j][r, sl]
                    if act:
                        s = 1.0 / (1.0 + jnp.exp(-s))
                    out_v[r, sl] = s
                return carry

            lax.fori_loop(0, BC, row, 0)
            pltpu.sync_copy(out_v, out_hbm.at[pl.ds(base, BC)])

    return run(idxc, table)


def kernel(features, knn_indices, W0, b0, W1, b1, W2, b2):
    n, _ = features.shape
    k = knn_indices.shape[1]
    nch = -(-n // (NW * BC))
    npad = NW * BC * nch

    x = jnp.pad(features, ((0, npad - n), (0, 0)))
    idx = jnp.pad(knn_indices, ((0, npad - n), (0, 0)))

    w2p = jnp.pad(W2, ((0, 16 - W2.shape[0]), (0, 0)))
    b2p = jnp.pad(b2, ((0, 0), (0, 16 - b2.shape[1])))

    kr = jnp.arange(k, dtype=jnp.int32)
    h = x
    for wgt, bias, act in ((W0, b0, True), (W1, b1, True), (w2p, b2p, False)):
        oc = wgt.shape[0]
        cin = h.shape[1]
        g = 128 // oc
        wk = wgt.reshape(oc, k, cin).transpose(1, 2, 0)
        wgrp = wk.reshape(k // g, g, cin, oc).transpose(0, 2, 1, 3)
        wgrp = wgrp.reshape(k // g, cin, 128)
        brow = jnp.pad(bias, ((0, 0), (0, 128 - oc)))
        idxa = idx * g + ((kr // g) * npad * g + kr % g)[None, :]
        idxc = idxa.reshape(NW, nch, BC, k).transpose(0, 1, 3, 2)
        y = _matmul_tables(h, wgrp, brow)
        table = y.reshape(npad * k, oc)
        h = _gather_sum(idxc, table, oc, act, npad)

    return h[:n, :3]
